# baseline (speedup 1.0000x reference)
import functools

import jax
import jax.numpy as jnp
from jax import lax
from jax.experimental import pallas as pl
from jax.experimental.pallas import tpu as pltpu
from jax.experimental.pallas import tpu_sc as plsc

NEG = -1e9
_PN = 384 * 384
_HALF = _PN // 2
_ROWS = _HALF // 128

_SC_MESH = plsc.VectorSubcoreMesh(core_axis_name="c", subcore_axis_name="s")
_SC_PARAMS = pltpu.CompilerParams(needs_layout_passes=False)


def _enc_kernel(gh_ref, gah_ref, text_ref, link_ref, conf_ref,
                enc_ref, stats_ref):
    conf = conf_ref[...]
    gh = gh_ref[...]
    gah = gah_ref[...]
    loss_g = (text_ref[...] - gh) ** 2 * conf
    loss_a = (link_ref[...] - gah) ** 2 * conf
    pos_g = gh >= 0.1
    pos_a = gah >= 0.1

    def rs(x):
        return jnp.sum(x, axis=(1, 2), keepdims=True).reshape(8, 1)

    enc_g = lax.bitcast_convert_type(jnp.where(pos_g, -loss_g, loss_g), jnp.int32)
    enc_a = lax.bitcast_convert_type(jnp.where(pos_a, -loss_a, loss_a), jnp.int32)
    enc_ref[...] = jnp.concatenate([enc_g, enc_a], axis=0).reshape(16, 1152, 128)

    pc = jnp.concatenate([rs(jnp.where(pos_g, 1.0, 0.0)),
                          rs(jnp.where(pos_a, 1.0, 0.0))], axis=0)
    sp = jnp.concatenate([rs(jnp.where(pos_g, loss_g, 0.0)),
                          rs(jnp.where(pos_a, loss_a, 0.0))], axis=0)
    sn = jnp.concatenate([rs(jnp.where(pos_g, 0.0, loss_g)),
                          rs(jnp.where(pos_a, 0.0, loss_a))], axis=0)
    lane = lax.broadcasted_iota(jnp.int32, (16, 128), 1)
    stats_ref[...] = (jnp.where(lane == 0, 3.0 * pc, 0.0)
                      + jnp.where(lane == 1, pc, 0.0)
                      + jnp.where(lane == 2, sp, 0.0)
                      + jnp.where(lane == 3, sn, 0.0))


def _sc_hist_sweep(data_v, level, b_n, b_a, cn0, cn1, ca0, ca1):
    ones = jnp.ones((16,), jnp.int32)

    def body(r, _):
        for j in range(8):
            v = data_v[r, pl.ds(j * 16, 16)]
            va = jnp.bitwise_and(v, 0x7FFFFFFF)
            if level == 0:
                idxn = lax.shift_right_logical(v, 20)
                idxa = lax.shift_right_logical(va, 20)
            elif level == 1:
                mn = lax.shift_right_logical(v, 20) == b_n
                ma = lax.shift_right_logical(va, 20) == b_a
                idxn = jnp.where(mn, jnp.bitwise_and(lax.shift_right_logical(v, 9), 0x7FF), 2048)
                idxa = jnp.where(ma, jnp.bitwise_and(lax.shift_right_logical(va, 9), 0x7FF), 2048)
            else:
                mn = lax.shift_right_logical(v, 9) == b_n
                ma = lax.shift_right_logical(va, 9) == b_a
                idxn = jnp.where(mn, jnp.bitwise_and(v, 0x1FF), 512)
                idxa = jnp.where(ma, jnp.bitwise_and(va, 0x1FF), 512)
            plsc.addupdate_scatter(cn0 if j % 2 == 0 else cn1, [idxn], ones)
            plsc.addupdate_scatter(ca0 if j % 2 == 0 else ca1, [idxa], ones)
        return 0

    lax.fori_loop(0, _ROWS, body, 0, unroll=False)


def _sc_merge(a_v, b_v, n):
    def body(i, _):
        a_v[pl.ds(i * 16, 16)] = a_v[pl.ds(i * 16, 16)] + b_v[pl.ds(i * 16, 16)]
        return 0
    lax.fori_loop(0, n // 16, body, 0, unroll=8)


def _sc_scan_hist(cnt_v, nbuck, k):
    nch = nbuck // 16

    def tbody(j, t):
        return t + jnp.sum(cnt_v[pl.ds(j * 16, 16)])

    total = lax.fori_loop(0, nch, tbody, jnp.int32(0), unroll=4)

    def sbody(j, carry):
        acc, bcnt, cle = carry
        h = cnt_v[pl.ds(j * 16, 16)]
        csum = plsc.cumsum(h)
        suffix = total - ((acc + csum) - h)
        m = suffix >= k
        bcnt = bcnt + jnp.sum(jnp.where(m, 1, 0))
        cle = cle + jnp.sum(jnp.where(m, h, 0))
        return (acc + jnp.sum(h), bcnt, cle)

    _, bcnt, cle = lax.fori_loop(
        0, nch, sbody, (jnp.int32(0), jnp.int32(0), jnp.int32(0)), unroll=4)
    return bcnt - 1, total - cle


def _sc_zero(ref, n):
    z = jnp.zeros((16,), jnp.int32)

    def body(i, _):
        ref[pl.ds(i * 16, 16)] = z
        return 0
    lax.fori_loop(0, n // 16, body, 0, unroll=8)


def _sc_exchange(local_v, partner_v, shared, s, slot, n):
    pltpu.sync_copy(local_v.at[pl.ds(0, n)], shared.at[s, slot, pl.ds(0, n)])
    plsc.subcore_barrier()
    sp = jnp.bitwise_xor(s, 1)
    pltpu.sync_copy(shared.at[sp, slot, pl.ds(0, n)], partner_v.at[pl.ds(0, n)])
    _sc_merge(local_v, partner_v, n)


def _select_kernel(enc_hbm, stats_hbm, out_hbm,
                   data_v, cn0, cn1, ca0, ca1, pn_v,
                   param_v, res_v, fx_v, px_v, shc, shf):
    c = lax.axis_index("c")
    s = lax.axis_index("s")
    row = c * 8 + lax.shift_right_logical(s, 1)
    half = jnp.bitwise_and(s, 1)

    pltpu.sync_copy(enc_hbm.at[row, pl.ds(half * _ROWS, _ROWS), :], data_v)
    pltpu.sync_copy(stats_hbm.at[row, pl.ds(0, 16)], param_v)
    pv = param_v[...]
    li = lax.broadcasted_iota(jnp.int32, (16,), 0)
    k_n = jnp.sum(jnp.where(li == 0, pv, 0.0)).astype(jnp.int32)
    k_a = jnp.int32(500)

    b_n = jnp.int32(0)
    b_a = jnp.int32(0)
    cgt_n = jnp.int32(0)
    cgt_a = jnp.int32(0)

    for level, nbuck in ((0, 2048), (1, 2048), (2, 512)):
        _sc_zero(cn0, nbuck)
        _sc_zero(cn1, nbuck)
        _sc_zero(ca0, nbuck)
        _sc_zero(ca1, nbuck)
        _sc_hist_sweep(data_v, level, b_n, b_a, cn0, cn1, ca0, ca1)
        _sc_merge(cn0, cn1, nbuck)
        _sc_merge(ca0, ca1, nbuck)
        _sc_exchange(cn0, pn_v, shc, s, 2 * level, nbuck)
        _sc_exchange(ca0, pn_v, shc, s, 2 * level + 1, nbuck)
        bn, cn = _sc_scan_hist(cn0, nbuck, k_n - cgt_n)
        ba, ca_ = _sc_scan_hist(ca0, nbuck, k_a - cgt_a)
        cgt_n = cgt_n + cn
        cgt_a = cgt_a + ca_
        if level == 0:
            b_n, b_a = bn, ba
        elif level == 1:
            b_n = b_n * 2048 + bn
            b_a = b_a * 2048 + ba
        else:
            b_n = b_n * 512 + bn
            b_a = b_a * 512 + ba
        plsc.subcore_barrier()

    zf = jnp.zeros((16,), jnp.float32)
    zi = jnp.zeros((16,), jnp.int32)

    def fbody(r, carry):
        cn_acc, sn_acc, ca_acc, sa_acc = carry
        for j in range(8):
            v = data_v[r, pl.ds(j * 16, 16)]
            va = jnp.bitwise_and(v, 0x7FFFFFFF)
            mn = v > b_n
            ma = va > b_a
            cn_acc = cn_acc + jnp.where(mn, 1, 0)
            sn_acc = sn_acc + jnp.where(mn, plsc.bitcast(v, jnp.float32), 0.0)
            ca_acc = ca_acc + jnp.where(ma, 1, 0)
            sa_acc = sa_acc + jnp.where(ma, plsc.bitcast(va, jnp.float32), 0.0)
        return cn_acc, sn_acc, ca_acc, sa_acc

    cn_acc, sn_acc, ca_acc, sa_acc = lax.fori_loop(
        0, _ROWS, fbody, (zi, zf, zi, zf), unroll=False)

    cn_s = jnp.sum(cn_acc).astype(jnp.float32)
    sn_s = jnp.sum(sn_acc)
    ca_s = jnp.sum(ca_acc).astype(jnp.float32)
    sa_s = jnp.sum(sa_acc)
    fx_v[...] = (jnp.where(li == 0, cn_s, 0.0)
                 + jnp.where(li == 1, sn_s, 0.0)
                 + jnp.where(li == 2, ca_s, 0.0)
                 + jnp.where(li == 3, sa_s, 0.0))
    pltpu.sync_copy(fx_v, shf.at[s, pl.ds(0, 16)])
    plsc.subcore_barrier()
    pltpu.sync_copy(shf.at[jnp.bitwise_xor(s, 1), pl.ds(0, 16)], px_v)
    tot = fx_v[...] + px_v[...]
    cgt_nf = jnp.sum(jnp.where(li == 0, tot, 0.0))
    sgt_n = jnp.sum(jnp.where(li == 1, tot, 0.0))
    cgt_af = jnp.sum(jnp.where(li == 2, tot, 0.0))
    sgt_a = jnp.sum(jnp.where(li == 3, tot, 0.0))

    tvn = plsc.bitcast(jnp.full((16,), 1, jnp.int32) * b_n, jnp.float32)
    tva = plsc.bitcast(jnp.full((16,), 1, jnp.int32) * b_a, jnp.float32)
    tk_n = sgt_n + (k_n.astype(jnp.float32) - cgt_nf) * tvn
    tk_a = sgt_a + (jnp.float32(500.0) - cgt_af) * tva
    res_v[...] = (jnp.where(li == 0, tk_n, 0.0) + jnp.where(li == 1, tk_a, 0.0))

    @pl.when(half == 0)
    def _():
        pltpu.sync_copy(res_v, out_hbm.at[row, pl.ds(0, 16)])


@functools.partial(
    pl.kernel, mesh=_SC_MESH, compiler_params=_SC_PARAMS,
    out_type=jax.ShapeDtypeStruct((16, 128), jnp.float32),
    scratch_types=[
        pltpu.VMEM((_ROWS, 128), jnp.int32),
        pltpu.VMEM((4096,), jnp.int32), pltpu.VMEM((4096,), jnp.int32),
        pltpu.VMEM((4096,), jnp.int32), pltpu.VMEM((4096,), jnp.int32),
        pltpu.VMEM((2048,), jnp.int32),
        pltpu.VMEM((16,), jnp.float32), pltpu.VMEM((16,), jnp.float32),
        pltpu.VMEM((16,), jnp.float32), pltpu.VMEM((16,), jnp.float32),
        pltpu.VMEM_SHARED((16, 6, 2048), jnp.int32),
        pltpu.VMEM_SHARED((16, 128), jnp.float32),
    ],
)
def _sc_select(enc_hbm, stats_hbm, out_hbm, *scratch):
    _select_kernel(enc_hbm, stats_hbm, out_hbm, *scratch)


def _rest_kernel(stats_ref, sc_ref, a_log_ref, p_log_ref, a_lab_ref,
                 p_lab_ref, lpt_ref, ext_ref, skip_ref, tlen_ref,
                 out_ref, lpe_ref):
    stats = stats_ref[...]
    scres = sc_ref[...]
    lane128 = lax.broadcasted_iota(jnp.int32, (16, 128), 1)

    def pick(x, j):
        return jnp.sum(jnp.where(lane128 == j, x, 0.0), axis=1, keepdims=True)

    k3 = pick(stats, 0)
    pcnt = pick(stats, 1)
    spos = pick(stats, 2)
    sneg = pick(stats, 3)
    tk = pick(scres, 0)
    t500 = pick(scres, 1)
    ncnt = float(_PN) - pcnt

    posi = spos / jnp.maximum(pcnt, 1.0)
    mean_neg = sneg / jnp.maximum(ncnt, 1.0)
    topk_neg = tk / jnp.maximum(k3, 1.0)
    nega = jnp.where(ncnt < k3, mean_neg, topk_neg)
    contrib = jnp.where(pcnt > 0, posi + nega, t500 / 500.0)
    saliency = jnp.sum(contrib) / 8.0

    def ce(logits, labels2d):
        n, cdim = logits.shape
        m = jnp.max(logits, axis=1, keepdims=True)
        ls = logits - m - jnp.log(jnp.sum(jnp.exp(logits - m), axis=1, keepdims=True))
        oh = lax.broadcasted_iota(jnp.int32, (n, cdim), 1) == labels2d
        return -jnp.sum(jnp.where(oh, ls, 0.0)) / float(n)

    direction = 0.5 * ce(p_log_ref[...], p_lab_ref[...]) + \
        0.5 * ce(a_log_ref[...], a_lab_ref[...])

    lpt = lpt_ref[...]
    ext = ext_ref[...]
    skipf = skip_ref[...]
    tlen = tlen_ref[...]
    N, T, C = lpt.shape
    L = ext.shape[1]

    oh = (ext[:, :, None] == lax.broadcasted_iota(jnp.int32, (N, L, C), 2))
    oh = oh.astype(jnp.float32)
    for n_i in range(N):
        lpe_ref[:, n_i, :] = lax.dot_general(
            lpt[n_i], oh[n_i], (((1,), (1,)), ((), ())),
            precision=lax.Precision.HIGHEST)

    li = lax.broadcasted_iota(jnp.int32, (N, L), 1)
    alpha0 = jnp.where(li <= 1, lpe_ref[0], NEG)

    def ctc_step(t, alpha):
        lp_t = lpe_ref[pl.ds(t, 1)].reshape(N, L)
        a1 = jnp.where(li >= 1, pltpu.roll(alpha, 1, 1), NEG)
        a2 = jnp.where(li >= 2, pltpu.roll(alpha, 2, 1), NEG)
        a2 = jnp.where(skipf > 0, a2, NEG)
        m = jnp.maximum(jnp.maximum(alpha, a1), a2)
        new = m + jnp.log(jnp.exp(alpha - m) + jnp.exp(a1 - m) + jnp.exp(a2 - m))
        new = new + lp_t
        return jnp.maximum(new, NEG)

    alpha = lax.fori_loop(1, T, ctc_step, alpha0)

    tl_i = tlen.astype(jnp.int32)
    i1 = jnp.clip(2 * tl_i, 0, L - 1)
    i2 = jnp.clip(2 * tl_i - 1, 0, L - 1)
    v1 = jnp.sum(jnp.where(li == i1, alpha, 0.0), axis=1, keepdims=True)
    v2 = jnp.sum(jnp.where(li == i2, alpha, 0.0), axis=1, keepdims=True)
    m = jnp.maximum(v1, v2)
    ll = m + jnp.log(jnp.exp(v1 - m) + jnp.exp(v2 - m))
    closs = -ll
    closs = jnp.where(closs < 1e8, closs, 0.0)
    recognition = 10.0 * jnp.mean(closs / jnp.maximum(tlen, 1.0))

    total = saliency + recognition
    lane = lax.broadcasted_iota(jnp.int32, (8, 128), 1)
    out = (jnp.where(lane == 0, total, 0.0) + jnp.where(lane == 1, saliency, 0.0)
           + jnp.where(lane == 2, direction, 0.0)
           + jnp.where(lane == 3, recognition, 0.0))
    out_ref[...] = out


@jax.jit
def _run(gh_label, gah_label, text_map, link_map, conf_map, a_logits, p_logits,
         a_label, p_label, log_probs, targets, target_lengths):
    N, S = targets.shape
    L = 2 * S + 1
    ext = jnp.zeros((N, L), dtype=targets.dtype)
    ext = ext.at[:, 1::2].set(targets)
    prev2 = jnp.concatenate(
        [jnp.full((N, 2), -1, dtype=ext.dtype), ext[:, :-2]], axis=1)
    allow_skip = ((ext != 0) & (ext != prev2)).astype(jnp.float32)
    lpt = jnp.transpose(log_probs, (1, 0, 2))
    tlen = target_lengths.astype(jnp.float32)[:, None]
    T = log_probs.shape[0]

    enc, stats = pl.pallas_call(
        _enc_kernel,
        out_shape=(jax.ShapeDtypeStruct((16, 1152, 128), jnp.int32),
                   jax.ShapeDtypeStruct((16, 128), jnp.float32)),
    )(gh_label, gah_label, text_map, link_map, conf_map)

    scres = _sc_select(enc, stats)

    out = pl.pallas_call(
        _rest_kernel,
        out_shape=jax.ShapeDtypeStruct((8, 128), jnp.float32),
        scratch_shapes=[pltpu.VMEM((T, N, L), jnp.float32)],
    )(stats, scres, a_logits, p_logits, a_label[:, None], p_label[:, None],
      lpt, ext, allow_skip, tlen)
    return out[0, 0], out[0, 1], out[0, 2], out[0, 3]


def kernel(gh_label, gah_label, text_map, link_map, conf_map, a_logits,
           p_logits, a_label, p_label, log_probs, targets, target_lengths):
    return _run(gh_label, gah_label, text_map, link_map, conf_map, a_logits,
                p_logits, a_label, p_label, log_probs, targets, target_lengths)


# speedup vs baseline: 1.6407x; 1.6407x over previous
import functools

import jax
import jax.numpy as jnp
from jax import lax
from jax.experimental import pallas as pl
from jax.experimental.pallas import tpu as pltpu
from jax.experimental.pallas import tpu_sc as plsc

NEG = -1e9
_PN = 384 * 384
_HALF = _PN // 2
_ROWS = _HALF // 128

_SC_MESH = plsc.VectorSubcoreMesh(core_axis_name="c", subcore_axis_name="s")
_SC_PARAMS = pltpu.CompilerParams(needs_layout_passes=False)


def _enc_kernel(gh_ref, gah_ref, text_ref, link_ref, conf_ref,
                enc_ref, stats_ref):
    conf = conf_ref[...]
    gh = gh_ref[...]
    gah = gah_ref[...]
    loss_g = (text_ref[...] - gh) ** 2 * conf
    loss_a = (link_ref[...] - gah) ** 2 * conf
    pos_g = gh >= 0.1
    pos_a = gah >= 0.1

    def rs(x):
        return jnp.sum(x, axis=(1, 2), keepdims=True).reshape(8, 1)

    enc_g = lax.bitcast_convert_type(jnp.where(pos_g, -loss_g, loss_g), jnp.int32)
    enc_a = lax.bitcast_convert_type(jnp.where(pos_a, -loss_a, loss_a), jnp.int32)
    enc_ref[...] = jnp.concatenate([enc_g, enc_a], axis=0).reshape(16, 1152, 128)

    pc = jnp.concatenate([rs(jnp.where(pos_g, 1.0, 0.0)),
                          rs(jnp.where(pos_a, 1.0, 0.0))], axis=0)
    sp = jnp.concatenate([rs(jnp.where(pos_g, loss_g, 0.0)),
                          rs(jnp.where(pos_a, loss_a, 0.0))], axis=0)
    sn = jnp.concatenate([rs(jnp.where(pos_g, 0.0, loss_g)),
                          rs(jnp.where(pos_a, 0.0, loss_a))], axis=0)
    lane = lax.broadcasted_iota(jnp.int32, (16, 128), 1)
    stats_ref[...] = (jnp.where(lane == 0, 3.0 * pc, 0.0)
                      + jnp.where(lane == 1, pc, 0.0)
                      + jnp.where(lane == 2, sp, 0.0)
                      + jnp.where(lane == 3, sn, 0.0))


def _sc_hist_sweep(data_v, level, b_n, b_a, cn0, cn1, ca0, ca1):
    ones = jnp.ones((16,), jnp.int32)
    dump = jnp.int32(2048) + lax.broadcasted_iota(jnp.int32, (16,), 0)

    def body(r, _):
        for j in range(8):
            v = data_v[r, pl.ds(j * 16, 16)]
            va = jnp.bitwise_and(v, 0x7FFFFFFF)
            if level == 0:
                mn = v >= 0
                idxn = jnp.where(mn, jnp.bitwise_and(lax.shift_right_logical(v, 20), 0x7FF), dump)
                idxa = lax.shift_right_logical(va, 20)
            elif level == 1:
                mn = lax.shift_right_logical(v, 20) == b_n
                ma = lax.shift_right_logical(va, 20) == b_a
                idxn = jnp.where(mn, jnp.bitwise_and(lax.shift_right_logical(v, 9), 0x7FF), dump)
                idxa = jnp.where(ma, jnp.bitwise_and(lax.shift_right_logical(va, 9), 0x7FF), dump)
            else:
                mn = lax.shift_right_logical(v, 9) == b_n
                ma = lax.shift_right_logical(va, 9) == b_a
                idxn = jnp.where(mn, jnp.bitwise_and(v, 0x1FF), dump)
                idxa = jnp.where(ma, jnp.bitwise_and(va, 0x1FF), dump)
            plsc.addupdate_scatter(cn0 if j % 2 == 0 else cn1, [idxn], ones)
            plsc.addupdate_scatter(ca0 if j % 2 == 0 else ca1, [idxa], ones)
        return 0

    lax.fori_loop(0, _ROWS, body, 0, unroll=False)


def _sc_merge(a_v, b_v, n):
    def body(i, _):
        a_v[pl.ds(i * 16, 16)] = a_v[pl.ds(i * 16, 16)] + b_v[pl.ds(i * 16, 16)]
        return 0
    lax.fori_loop(0, n // 16, body, 0, unroll=8)


def _sc_scan_hist(cnt_v, nbuck, k):
    nch = nbuck // 16

    def tbody(j, t):
        return t + jnp.sum(cnt_v[pl.ds(j * 16, 16)])

    total = lax.fori_loop(0, nch, tbody, jnp.int32(0), unroll=4)

    def sbody(j, carry):
        acc, bcnt, cle = carry
        h = cnt_v[pl.ds(j * 16, 16)]
        csum = plsc.cumsum(h)
        suffix = total - ((acc + csum) - h)
        m = suffix >= k
        bcnt = bcnt + jnp.sum(jnp.where(m, 1, 0))
        cle = cle + jnp.sum(jnp.where(m, h, 0))
        return (acc + jnp.sum(h), bcnt, cle)

    _, bcnt, cle = lax.fori_loop(
        0, nch, sbody, (jnp.int32(0), jnp.int32(0), jnp.int32(0)), unroll=4)
    return bcnt - 1, total - cle


def _sc_zero(ref, n):
    z = jnp.zeros((16,), jnp.int32)

    def body(i, _):
        ref[pl.ds(i * 16, 16)] = z
        return 0
    lax.fori_loop(0, n // 16, body, 0, unroll=8)


def _sc_exchange(local_v, partner_v, shared, s, slot, n):
    pltpu.sync_copy(local_v.at[pl.ds(0, n)], shared.at[s, slot, pl.ds(0, n)])
    plsc.subcore_barrier()
    sp = jnp.bitwise_xor(s, 1)
    pltpu.sync_copy(shared.at[sp, slot, pl.ds(0, n)], partner_v.at[pl.ds(0, n)])
    _sc_merge(local_v, partner_v, n)


def _select_kernel(enc_hbm, stats_hbm, out_hbm,
                   data_v, cn0, cn1, ca0, ca1, pn_v,
                   param_v, res_v, fx_v, px_v, shc, shf):
    c = lax.axis_index("c")
    s = lax.axis_index("s")
    row = c * 8 + lax.shift_right_logical(s, 1)
    half = jnp.bitwise_and(s, 1)

    pltpu.sync_copy(enc_hbm.at[row, pl.ds(half * _ROWS, _ROWS), :], data_v)
    pltpu.sync_copy(stats_hbm.at[row, pl.ds(0, 16)], param_v)
    pv = param_v[...]
    li = lax.broadcasted_iota(jnp.int32, (16,), 0)
    k_n = jnp.sum(jnp.where(li == 0, pv, 0.0)).astype(jnp.int32)
    k_a = jnp.int32(500)

    b_n = jnp.int32(0)
    b_a = jnp.int32(0)
    cgt_n = jnp.int32(0)
    cgt_a = jnp.int32(0)

    for level, nbuck in ((0, 2048), (1, 2048), (2, 512)):
        _sc_zero(cn0, nbuck)
        _sc_zero(cn1, nbuck)
        _sc_zero(ca0, nbuck)
        _sc_zero(ca1, nbuck)
        _sc_hist_sweep(data_v, level, b_n, b_a, cn0, cn1, ca0, ca1)
        _sc_merge(cn0, cn1, nbuck)
        _sc_merge(ca0, ca1, nbuck)
        _sc_exchange(cn0, pn_v, shc, s, 2 * level, nbuck)
        _sc_exchange(ca0, pn_v, shc, s, 2 * level + 1, nbuck)
        bn, cn = _sc_scan_hist(cn0, nbuck, k_n - cgt_n)
        ba, ca_ = _sc_scan_hist(ca0, nbuck, k_a - cgt_a)
        cgt_n = cgt_n + cn
        cgt_a = cgt_a + ca_
        if level == 0:
            b_n, b_a = bn, ba
        elif level == 1:
            b_n = b_n * 2048 + bn
            b_a = b_a * 2048 + ba
        else:
            b_n = b_n * 512 + bn
            b_a = b_a * 512 + ba
        plsc.subcore_barrier()

    zf = jnp.zeros((16,), jnp.float32)
    zi = jnp.zeros((16,), jnp.int32)

    def fbody(r, carry):
        cn_acc, sn_acc, ca_acc, sa_acc = carry
        for j in range(8):
            v = data_v[r, pl.ds(j * 16, 16)]
            va = jnp.bitwise_and(v, 0x7FFFFFFF)
            mn = v > b_n
            ma = va > b_a
            cn_acc = cn_acc + jnp.where(mn, 1, 0)
            sn_acc = sn_acc + jnp.where(mn, plsc.bitcast(v, jnp.float32), 0.0)
            ca_acc = ca_acc + jnp.where(ma, 1, 0)
            sa_acc = sa_acc + jnp.where(ma, plsc.bitcast(va, jnp.float32), 0.0)
        return cn_acc, sn_acc, ca_acc, sa_acc

    cn_acc, sn_acc, ca_acc, sa_acc = lax.fori_loop(
        0, _ROWS, fbody, (zi, zf, zi, zf), unroll=False)

    cn_s = jnp.sum(cn_acc).astype(jnp.float32)
    sn_s = jnp.sum(sn_acc)
    ca_s = jnp.sum(ca_acc).astype(jnp.float32)
    sa_s = jnp.sum(sa_acc)
    fx_v[...] = (jnp.where(li == 0, cn_s, 0.0)
                 + jnp.where(li == 1, sn_s, 0.0)
                 + jnp.where(li == 2, ca_s, 0.0)
                 + jnp.where(li == 3, sa_s, 0.0))
    pltpu.sync_copy(fx_v, shf.at[s, pl.ds(0, 16)])
    plsc.subcore_barrier()
    pltpu.sync_copy(shf.at[jnp.bitwise_xor(s, 1), pl.ds(0, 16)], px_v)
    tot = fx_v[...] + px_v[...]
    cgt_nf = jnp.sum(jnp.where(li == 0, tot, 0.0))
    sgt_n = jnp.sum(jnp.where(li == 1, tot, 0.0))
    cgt_af = jnp.sum(jnp.where(li == 2, tot, 0.0))
    sgt_a = jnp.sum(jnp.where(li == 3, tot, 0.0))

    tvn = plsc.bitcast(jnp.full((16,), 1, jnp.int32) * b_n, jnp.float32)
    tva = plsc.bitcast(jnp.full((16,), 1, jnp.int32) * b_a, jnp.float32)
    tk_n = sgt_n + (k_n.astype(jnp.float32) - cgt_nf) * tvn
    tk_a = sgt_a + (jnp.float32(500.0) - cgt_af) * tva
    res_v[...] = (jnp.where(li == 0, tk_n, 0.0) + jnp.where(li == 1, tk_a, 0.0))

    @pl.when(half == 0)
    def _():
        pltpu.sync_copy(res_v, out_hbm.at[row, pl.ds(0, 16)])


@functools.partial(
    pl.kernel, mesh=_SC_MESH, compiler_params=_SC_PARAMS,
    out_type=jax.ShapeDtypeStruct((16, 128), jnp.float32),
    scratch_types=[
        pltpu.VMEM((_ROWS, 128), jnp.int32),
        pltpu.VMEM((4096,), jnp.int32), pltpu.VMEM((4096,), jnp.int32),
        pltpu.VMEM((4096,), jnp.int32), pltpu.VMEM((4096,), jnp.int32),
        pltpu.VMEM((2048,), jnp.int32),
        pltpu.VMEM((16,), jnp.float32), pltpu.VMEM((16,), jnp.float32),
        pltpu.VMEM((16,), jnp.float32), pltpu.VMEM((16,), jnp.float32),
        pltpu.VMEM_SHARED((16, 6, 2048), jnp.int32),
        pltpu.VMEM_SHARED((16, 128), jnp.float32),
    ],
)
def _sc_select(enc_hbm, stats_hbm, out_hbm, *scratch):
    _select_kernel(enc_hbm, stats_hbm, out_hbm, *scratch)


def _rest_kernel(stats_ref, sc_ref, a_log_ref, p_log_ref, a_lab_ref,
                 p_lab_ref, lpt_ref, ext_ref, skip_ref, tlen_ref,
                 out_ref, lpe_ref):
    stats = stats_ref[...]
    scres = sc_ref[...]
    lane128 = lax.broadcasted_iota(jnp.int32, (16, 128), 1)

    def pick(x, j):
        return jnp.sum(jnp.where(lane128 == j, x, 0.0), axis=1, keepdims=True)

    k3 = pick(stats, 0)
    pcnt = pick(stats, 1)
    spos = pick(stats, 2)
    sneg = pick(stats, 3)
    tk = pick(scres, 0)
    t500 = pick(scres, 1)
    ncnt = float(_PN) - pcnt

    posi = spos / jnp.maximum(pcnt, 1.0)
    mean_neg = sneg / jnp.maximum(ncnt, 1.0)
    topk_neg = tk / jnp.maximum(k3, 1.0)
    nega = jnp.where(ncnt < k3, mean_neg, topk_neg)
    contrib = jnp.where(pcnt > 0, posi + nega, t500 / 500.0)
    saliency = jnp.sum(contrib) / 8.0

    def ce(logits, labels2d):
        n, cdim = logits.shape
        m = jnp.max(logits, axis=1, keepdims=True)
        ls = logits - m - jnp.log(jnp.sum(jnp.exp(logits - m), axis=1, keepdims=True))
        oh = lax.broadcasted_iota(jnp.int32, (n, cdim), 1) == labels2d
        return -jnp.sum(jnp.where(oh, ls, 0.0)) / float(n)

    direction = 0.5 * ce(p_log_ref[...], p_lab_ref[...]) + \
        0.5 * ce(a_log_ref[...], a_lab_ref[...])

    lpt = lpt_ref[...]
    ext = ext_ref[...]
    skipf = skip_ref[...]
    tlen = tlen_ref[...]
    N, T, C = lpt.shape
    L = ext.shape[1]

    oh = (ext[:, :, None] == lax.broadcasted_iota(jnp.int32, (N, L, C), 2))
    oh = oh.astype(jnp.float32)
    for n_i in range(N):
        lpe_ref[:, n_i, :] = lax.dot_general(
            lpt[n_i], oh[n_i], (((1,), (1,)), ((), ())),
            precision=lax.Precision.HIGHEST)

    li = lax.broadcasted_iota(jnp.int32, (N, L), 1)
    alpha0 = jnp.where(li <= 1, lpe_ref[0], NEG)

    def ctc_step(t, alpha):
        lp_t = lpe_ref[pl.ds(t, 1)].reshape(N, L)
        a1 = jnp.where(li >= 1, pltpu.roll(alpha, 1, 1), NEG)
        a2 = jnp.where(li >= 2, pltpu.roll(alpha, 2, 1), NEG)
        a2 = jnp.where(skipf > 0, a2, NEG)
        m = jnp.maximum(jnp.maximum(alpha, a1), a2)
        new = m + jnp.log(jnp.exp(alpha - m) + jnp.exp(a1 - m) + jnp.exp(a2 - m))
        new = new + lp_t
        return jnp.maximum(new, NEG)

    alpha = lax.fori_loop(1, T, ctc_step, alpha0)

    tl_i = tlen.astype(jnp.int32)
    i1 = jnp.clip(2 * tl_i, 0, L - 1)
    i2 = jnp.clip(2 * tl_i - 1, 0, L - 1)
    v1 = jnp.sum(jnp.where(li == i1, alpha, 0.0), axis=1, keepdims=True)
    v2 = jnp.sum(jnp.where(li == i2, alpha, 0.0), axis=1, keepdims=True)
    m = jnp.maximum(v1, v2)
    ll = m + jnp.log(jnp.exp(v1 - m) + jnp.exp(v2 - m))
    closs = -ll
    closs = jnp.where(closs < 1e8, closs, 0.0)
    recognition = 10.0 * jnp.mean(closs / jnp.maximum(tlen, 1.0))

    total = saliency + recognition
    lane = lax.broadcasted_iota(jnp.int32, (8, 128), 1)
    out = (jnp.where(lane == 0, total, 0.0) + jnp.where(lane == 1, saliency, 0.0)
           + jnp.where(lane == 2, direction, 0.0)
           + jnp.where(lane == 3, recognition, 0.0))
    out_ref[...] = out


@jax.jit
def _run(gh_label, gah_label, text_map, link_map, conf_map, a_logits, p_logits,
         a_label, p_label, log_probs, targets, target_lengths):
    N, S = targets.shape
    L = 2 * S + 1
    ext = jnp.zeros((N, L), dtype=targets.dtype)
    ext = ext.at[:, 1::2].set(targets)
    prev2 = jnp.concatenate(
        [jnp.full((N, 2), -1, dtype=ext.dtype), ext[:, :-2]], axis=1)
    allow_skip = ((ext != 0) & (ext != prev2)).astype(jnp.float32)
    lpt = jnp.transpose(log_probs, (1, 0, 2))
    tlen = target_lengths.astype(jnp.float32)[:, None]
    T = log_probs.shape[0]

    enc, stats = pl.pallas_call(
        _enc_kernel,
        out_shape=(jax.ShapeDtypeStruct((16, 1152, 128), jnp.int32),
                   jax.ShapeDtypeStruct((16, 128), jnp.float32)),
    )(gh_label, gah_label, text_map, link_map, conf_map)

    scres = _sc_select(enc, stats)

    out = pl.pallas_call(
        _rest_kernel,
        out_shape=jax.ShapeDtypeStruct((8, 128), jnp.float32),
        scratch_shapes=[pltpu.VMEM((T, N, L), jnp.float32)],
    )(stats, scres, a_logits, p_logits, a_label[:, None], p_label[:, None],
      lpt, ext, allow_skip, tlen)
    return out[0, 0], out[0, 1], out[0, 2], out[0, 3]


def kernel(gh_label, gah_label, text_map, link_map, conf_map, a_logits,
           p_logits, a_label, p_label, log_probs, targets, target_lengths):
    return _run(gh_label, gah_label, text_map, link_map, conf_map, a_logits,
                p_logits, a_label, p_label, log_probs, targets, target_lengths)
